# baseline (speedup 1.0000x reference)
import functools

import jax
import jax.numpy as jnp
from jax import lax
from jax.experimental import pallas as pl
from jax.experimental.pallas import tpu as pltpu
from jax.experimental.pallas import tpu_sc as plsc

DIM = 64
MAX_NORM = 1.0


_P = 1024


def _renorm_body(tt_ref, out_ref):
    x = tt_ref[...]
    norm = jnp.sqrt(jnp.sum(x * x, axis=0, keepdims=True))
    scale = jnp.where(norm > MAX_NORM, MAX_NORM / (norm + 1e-7), 1.0)
    y = x * scale
    out_ref[...] = jnp.concatenate([y[:, :_P].T, y[:, _P:].T], axis=1)


def _renorm_flat(table):
    vocab = table.shape[0]
    g = (vocab + 2 * _P - 1) // (2 * _P)
    sf = pl.pallas_call(
        _renorm_body,
        grid=(g,),
        in_specs=[pl.BlockSpec((DIM, 2 * _P), lambda i: (0, i))],
        out_specs=pl.BlockSpec((_P, 2 * DIM), lambda i: (i, 0)),
        out_shape=jax.ShapeDtypeStruct((g * _P, 2 * DIM), jnp.float32),
    )(table.T)
    return sf.reshape(g * 2 * _P, DIM)


_CHUNK = 128
_NG = 5
_NT = 3
_LAG = 3


@functools.cache
def _make_gather(bsz, seq):
    info = plsc.get_sparse_core_info()
    nc, ns = info.num_cores, info.num_subcores
    nw = nc * ns
    bblks = bsz // _CHUNK
    per_w = bblks * seq // nw
    assert per_w * nw == bblks * seq and bblks * _CHUNK == bsz
    assert bblks & (bblks - 1) == 0
    sh_l = bblks.bit_length() - 1
    sh_b = _CHUNK.bit_length() - 1
    mesh = plsc.VectorSubcoreMesh(core_axis_name="c", subcore_axis_name="s")

    @functools.partial(
        pl.kernel,
        mesh=mesh,
        compiler_params=pltpu.CompilerParams(
            use_tc_tiling_on_sc=False, needs_layout_passes=False),
        out_type=jax.ShapeDtypeStruct((seq, DIM, bsz), jnp.float32),
        scratch_types=(
            [pltpu.VMEM((per_w, _CHUNK), jnp.int32)]
            + [pltpu.VMEM((_CHUNK, DIM), jnp.float32) for _ in range(_NG)]
            + [pltpu.VMEM((DIM, _CHUNK), jnp.float32) for _ in range(_NT)]
            + [pltpu.SemaphoreType.DMA for _ in range(_NG + _NT)]
        ),
    )
    def gather_k(tab_hbm, idx_hbm, out_hbm, idx_v, *rest):
        gbufs = rest[:_NG]
        tbufs = rest[_NG:_NG + _NT]
        gsems = rest[_NG + _NT:2 * _NG + _NT]
        wsems = rest[2 * _NG + _NT:]
        wid = lax.axis_index("s") * nc + lax.axis_index("c")
        ubase = wid * per_w
        pltpu.sync_copy(idx_hbm.at[wid], idx_v)

        @pl.loop(0, per_w)
        def _(j):
            for k in range(_CHUNK // 16):
                v = idx_v[j, pl.ds(k * 16, 16)]
                g = jnp.right_shift(v, 11)
                h = jnp.bitwise_and(jnp.right_shift(v, 10), 1)
                i2 = jnp.bitwise_and(v, 1023)
                idx_v[j, pl.ds(k * 16, 16)] = (
                    jnp.left_shift(g, 11) + jnp.left_shift(i2, 1) + h)

        iotas = [lax.iota(jnp.int32, 16) + 16 * k for k in range(DIM // 16)]

        def transpose_chunk(gb, tb):
            @pl.loop(0, _CHUNK)
            def _(r):
                col = jnp.full((16,), r, jnp.int32)
                for k in range(DIM // 16):
                    v = gb[r, pl.ds(16 * k, 16)]
                    plsc.store_scatter(tb, [iotas[k], col], v)

        hg = [None] * _NG
        hw = [None] * _NT
        for j in range(per_w + _LAG):
            if j < per_w:
                hg[j % _NG] = pltpu.async_copy(
                    tab_hbm.at[idx_v.at[j]], gbufs[j % _NG], gsems[j % _NG])
            i = j - _LAG
            if 0 <= i < per_w:
                hg[i % _NG].wait()
                if i >= _NT:
                    hw[i % _NT].wait()
                transpose_chunk(gbufs[i % _NG], tbufs[i % _NT])
                u = ubase + i
                li = jnp.right_shift(u, sh_l)
                b0 = pl.multiple_of(
                    jnp.left_shift(jnp.bitwise_and(u, bblks - 1), sh_b),
                    _CHUNK)
                hw[i % _NT] = pltpu.async_copy(
                    tbufs[i % _NT],
                    out_hbm.at[li, :, pl.ds(b0, _CHUNK)],
                    wsems[i % _NT])
        for i in range(max(0, per_w - _NT), per_w):
            hw[i % _NT].wait()

    return gather_k


def kernel(doc, table):
    b, l = doc.shape
    flat = _renorm_flat(table)
    nw = 32
    idx3d = doc.T.reshape(nw, b * l // (_CHUNK * nw), _CHUNK)
    out = _make_gather(b, l)(flat, idx3d)
    return jnp.transpose(out, (2, 0, 1))


# speedup vs baseline: 2.1780x; 2.1780x over previous
import functools

import jax
import jax.numpy as jnp
from jax import lax
from jax.experimental import pallas as pl
from jax.experimental.pallas import tpu as pltpu
from jax.experimental.pallas import tpu_sc as plsc

DIM = 64
MAX_NORM = 1.0


_P = 1024


def _renorm_body(tt_ref, out_ref):
    x = tt_ref[...]
    norm = jnp.sqrt(jnp.sum(x * x, axis=0, keepdims=True))
    scale = jnp.where(norm > MAX_NORM, MAX_NORM / (norm + 1e-7), 1.0)
    y = x * scale
    out_ref[...] = jnp.concatenate([y[:, :_P].T, y[:, _P:].T], axis=1)


def _renorm_flat(table):
    vocab = table.shape[0]
    g = (vocab + 2 * _P - 1) // (2 * _P)
    sf = pl.pallas_call(
        _renorm_body,
        grid=(g,),
        in_specs=[pl.BlockSpec((DIM, 2 * _P), lambda i: (0, i))],
        out_specs=pl.BlockSpec((_P, 2 * DIM), lambda i: (i, 0)),
        out_shape=jax.ShapeDtypeStruct((g * _P, 2 * DIM), jnp.float32),
    )(table.T)
    return sf.reshape(g * 2 * _P, DIM)


_CHUNK = 128
_NBUF = 6
_LAG = _NBUF // 2


@functools.cache
def _make_gather(bsz, seq):
    info = plsc.get_sparse_core_info()
    nc, ns = info.num_cores, info.num_subcores
    nw = nc * ns
    bblks = bsz // _CHUNK
    half = bsz // 2
    per_w = bblks * seq // nw
    assert per_w * nw == bblks * seq and bblks * _CHUNK == bsz
    assert bblks & (bblks - 1) == 0
    sh_l = bblks.bit_length() - 1
    sh_b = _CHUNK.bit_length() - 1
    mesh = plsc.VectorSubcoreMesh(core_axis_name="c", subcore_axis_name="s")

    @functools.partial(
        pl.kernel,
        mesh=mesh,
        compiler_params=pltpu.CompilerParams(
            use_tc_tiling_on_sc=False, needs_layout_passes=False),
        out_type=jax.ShapeDtypeStruct((seq * half, 2 * DIM), jnp.float32),
        scratch_types=(
            [pltpu.VMEM((per_w, _CHUNK), jnp.int32)]
            + [pltpu.VMEM((_CHUNK, DIM), jnp.float32) for _ in range(_NBUF)]
            + [pltpu.SemaphoreType.DMA for _ in range(2 * _NBUF)]
        ),
    )
    def gather_k(tab_hbm, idx_hbm, out_hbm, idx_v, *rest):
        bufs = rest[:_NBUF]
        gsems = rest[_NBUF:2 * _NBUF]
        wsems = rest[2 * _NBUF:]
        wid = lax.axis_index("s") * nc + lax.axis_index("c")
        ubase = wid * per_w
        pltpu.sync_copy(idx_hbm.at[wid], idx_v)

        @pl.loop(0, per_w)
        def _(j):
            for k in range(_CHUNK // 16):
                v = idx_v[j, pl.ds(k * 16, 16)]
                g = jnp.right_shift(v, 11)
                h = jnp.bitwise_and(jnp.right_shift(v, 10), 1)
                i2 = jnp.bitwise_and(v, 1023)
                idx_v[j, pl.ds(k * 16, 16)] = (
                    jnp.left_shift(g, 11) + jnp.left_shift(i2, 1) + h)

        hg = [None] * _NBUF
        hw = [None] * _NBUF
        for j in range(per_w + _LAG):
            if j < per_w:
                b = j % _NBUF
                if j >= _NBUF:
                    hw[b].wait()
                hg[b] = pltpu.async_copy(
                    tab_hbm.at[idx_v.at[j]], bufs[b], gsems[b])
            i = j - _LAG
            if 0 <= i < per_w:
                bi = i % _NBUF
                hg[bi].wait()
                u = ubase + i
                li = jnp.right_shift(u, sh_l)
                b0 = jnp.left_shift(jnp.bitwise_and(u, bblks - 1), sh_b)
                hh = jnp.right_shift(b0, half.bit_length() - 1)
                row0 = pl.multiple_of(
                    li * half + jnp.bitwise_and(b0, half - 1), _CHUNK)
                col0 = pl.multiple_of(hh * DIM, DIM)
                hw[bi] = pltpu.async_copy(
                    bufs[bi],
                    out_hbm.at[pl.ds(row0, _CHUNK), pl.ds(col0, DIM)],
                    wsems[bi])
        for i in range(max(0, per_w - _NBUF), per_w):
            hw[i % _NBUF].wait()

    return gather_k


def _plane_body(f_ref, out_ref):
    x = f_ref[...]
    out_ref[...] = jnp.concatenate(
        [x[:, :DIM].T, x[:, DIM:].T], axis=1)[None]


def _plane_transpose(f, seq, bsz):
    half = bsz // 2
    return pl.pallas_call(
        _plane_body,
        grid=(seq,),
        in_specs=[pl.BlockSpec((half, 2 * DIM), lambda i: (i, 0))],
        out_specs=pl.BlockSpec((1, DIM, bsz), lambda i: (i, 0, 0)),
        out_shape=jax.ShapeDtypeStruct((seq, DIM, bsz), jnp.float32),
    )(f)


def kernel(doc, table):
    b, l = doc.shape
    flat = _renorm_flat(table)
    nw = 32
    idx3d = doc.T.reshape(nw, b * l // (_CHUNK * nw), _CHUNK)
    f = _make_gather(b, l)(flat, idx3d)
    out = _plane_transpose(f, l, b)
    return jnp.transpose(out, (2, 0, 1))


# speedup vs baseline: 2.2811x; 1.0474x over previous
import functools

import jax
import jax.numpy as jnp
from jax import lax
from jax.experimental import pallas as pl
from jax.experimental.pallas import tpu as pltpu
from jax.experimental.pallas import tpu_sc as plsc

DIM = 64
MAX_NORM = 1.0


_P = 1024


def _renorm_body(tt_ref, out_ref):
    x = tt_ref[...]
    norm = jnp.sqrt(jnp.sum(x * x, axis=0, keepdims=True))
    scale = jnp.where(norm > MAX_NORM, MAX_NORM / (norm + 1e-7), 1.0)
    y = x * scale
    out_ref[...] = jnp.concatenate([y[:, :_P].T, y[:, _P:].T], axis=1)


def _renorm_flat(table):
    vocab = table.shape[0]
    g = (vocab + 2 * _P - 1) // (2 * _P)
    sf = pl.pallas_call(
        _renorm_body,
        grid=(g,),
        in_specs=[pl.BlockSpec((DIM, 2 * _P), lambda i: (0, i))],
        out_specs=pl.BlockSpec((_P, 2 * DIM), lambda i: (i, 0)),
        out_shape=jax.ShapeDtypeStruct((g * _P, 2 * DIM), jnp.float32),
    )(table.T)
    return sf.reshape(g * 2 * _P, DIM)


_CHUNK = 128
_NBUF = 6
_LAG = _NBUF // 2


@functools.cache
def _make_gather(bsz, seq):
    info = plsc.get_sparse_core_info()
    nc, ns = info.num_cores, info.num_subcores
    nw = nc * ns
    bblks = bsz // _CHUNK
    half = bsz // 2
    per_w = bblks * seq // nw
    assert per_w * nw == bblks * seq and bblks * _CHUNK == bsz
    assert bblks & (bblks - 1) == 0
    sh_l = bblks.bit_length() - 1
    sh_b = _CHUNK.bit_length() - 1
    mesh = plsc.VectorSubcoreMesh(core_axis_name="c", subcore_axis_name="s")

    @functools.partial(
        pl.kernel,
        mesh=mesh,
        compiler_params=pltpu.CompilerParams(
            use_tc_tiling_on_sc=False, needs_layout_passes=False),
        out_type=jax.ShapeDtypeStruct((seq * half, 2 * DIM), jnp.float32),
        scratch_types=(
            [pltpu.VMEM((per_w, _CHUNK), jnp.int32)]
            + [pltpu.VMEM((_CHUNK, DIM), jnp.float32) for _ in range(_NBUF)]
            + [pltpu.SemaphoreType.DMA for _ in range(2 * _NBUF)]
        ),
    )
    def gather_k(tab_hbm, idx_hbm, out_hbm, idx_v, *rest):
        bufs = rest[:_NBUF]
        gsems = rest[_NBUF:2 * _NBUF]
        wsems = rest[2 * _NBUF:]
        wid = lax.axis_index("s") * nc + lax.axis_index("c")
        ubase = wid * per_w
        pltpu.sync_copy(idx_hbm.at[wid], idx_v)

        @pl.loop(0, per_w)
        def _(j):
            for k in range(_CHUNK // 16):
                v = idx_v[j, pl.ds(k * 16, 16)]
                g = jnp.right_shift(v, 11)
                h = jnp.bitwise_and(jnp.right_shift(v, 10), 1)
                i2 = jnp.bitwise_and(v, 1023)
                idx_v[j, pl.ds(k * 16, 16)] = (
                    jnp.left_shift(g, 11) + jnp.left_shift(i2, 1) + h)

        hg = [None] * _NBUF
        hw = [None] * _NBUF
        for j in range(per_w + _LAG):
            if j < per_w:
                b = j % _NBUF
                if j >= _NBUF:
                    hw[b].wait()
                hg[b] = pltpu.async_copy(
                    tab_hbm.at[idx_v.at[j]], bufs[b], gsems[b])
            i = j - _LAG
            if 0 <= i < per_w:
                bi = i % _NBUF
                hg[bi].wait()
                u = ubase + i
                li = jnp.right_shift(u, sh_l)
                b0 = jnp.left_shift(jnp.bitwise_and(u, bblks - 1), sh_b)
                hh = jnp.right_shift(b0, half.bit_length() - 1)
                row0 = pl.multiple_of(
                    li * half + jnp.bitwise_and(b0, half - 1), _CHUNK)
                col0 = pl.multiple_of(hh * DIM, DIM)
                hw[bi] = pltpu.async_copy(
                    bufs[bi],
                    out_hbm.at[pl.ds(row0, _CHUNK), pl.ds(col0, DIM)],
                    wsems[bi])
        for i in range(max(0, per_w - _NBUF), per_w):
            hw[i % _NBUF].wait()

    return gather_k


def _eye(n):
    return (jax.lax.broadcasted_iota(jnp.int32, (n, n), 0)
            == jax.lax.broadcasted_iota(jnp.int32, (n, n), 1)
            ).astype(jnp.float32)


def _plane_body(f_ref, out_ref):
    x = f_ref[...]
    ident = _eye(DIM)
    dn = (((1,), (1,)), ((), ()))
    a = jax.lax.dot_general(ident, x[:, :DIM], dn,
                            preferred_element_type=jnp.float32)
    b = jax.lax.dot_general(ident, x[:, DIM:], dn,
                            preferred_element_type=jnp.float32)
    out_ref[...] = jnp.concatenate([a, b], axis=1)[None]


def _plane_transpose(f, seq, bsz):
    half = bsz // 2
    return pl.pallas_call(
        _plane_body,
        grid=(seq,),
        in_specs=[pl.BlockSpec((half, 2 * DIM), lambda i: (i, 0))],
        out_specs=pl.BlockSpec((1, DIM, bsz), lambda i: (i, 0, 0)),
        out_shape=jax.ShapeDtypeStruct((seq, DIM, bsz), jnp.float32),
    )(f)


def kernel(doc, table):
    b, l = doc.shape
    flat = _renorm_flat(table)
    nw = 32
    idx3d = doc.T.reshape(nw, b * l // (_CHUNK * nw), _CHUNK)
    f = _make_gather(b, l)(flat, idx3d)
    out = _plane_transpose(f, l, b)
    return jnp.transpose(out, (2, 0, 1))


# speedup vs baseline: 2.7202x; 1.1925x over previous
import functools

import jax
import jax.numpy as jnp
from jax import lax
from jax.experimental import pallas as pl
from jax.experimental.pallas import tpu as pltpu
from jax.experimental.pallas import tpu_sc as plsc

DIM = 64
MAX_NORM = 1.0


_P = 2048


def _renorm_body(tt_ref, out_ref):
    x = tt_ref[...]
    norm = jnp.sqrt(jnp.sum(x * x, axis=0, keepdims=True))
    scale = jnp.where(norm > MAX_NORM, MAX_NORM / (norm + 1e-7), 1.0)
    y = x * scale
    out_ref[...] = jnp.concatenate([y[:, :_P].T, y[:, _P:].T], axis=1)


def _renorm_flat(table):
    vocab = table.shape[0]
    g = (vocab + 2 * _P - 1) // (2 * _P)
    sf = pl.pallas_call(
        _renorm_body,
        grid=(g,),
        in_specs=[pl.BlockSpec((DIM, 2 * _P), lambda i: (0, i))],
        out_specs=pl.BlockSpec((_P, 2 * DIM), lambda i: (i, 0)),
        out_shape=jax.ShapeDtypeStruct((g * _P, 2 * DIM), jnp.float32),
    )(table.T)
    return sf.reshape(g * 2 * _P, DIM)


_CHUNK = 128
_NBUF = 6
_LAG = _NBUF // 2


@functools.cache
def _make_gather(bsz, seq):
    info = plsc.get_sparse_core_info()
    nc, ns = info.num_cores, info.num_subcores
    nw = nc * ns
    bblks = bsz // _CHUNK
    half = bsz // 2
    per_w = bblks * seq // nw
    assert per_w * nw == bblks * seq and bblks * _CHUNK == bsz
    assert bblks & (bblks - 1) == 0
    sh_l = bblks.bit_length() - 1
    sh_b = _CHUNK.bit_length() - 1
    mesh = plsc.VectorSubcoreMesh(core_axis_name="c", subcore_axis_name="s")

    @functools.partial(
        pl.kernel,
        mesh=mesh,
        compiler_params=pltpu.CompilerParams(
            use_tc_tiling_on_sc=False, needs_layout_passes=False),
        out_type=jax.ShapeDtypeStruct((seq * half, 2 * DIM), jnp.float32),
        scratch_types=(
            [pltpu.VMEM((per_w, _CHUNK), jnp.int32)]
            + [pltpu.VMEM((_CHUNK, DIM), jnp.float32) for _ in range(_NBUF)]
            + [pltpu.SemaphoreType.DMA for _ in range(2 * _NBUF)]
        ),
    )
    def gather_k(tab_hbm, idx_hbm, out_hbm, idx_v, *rest):
        bufs = rest[:_NBUF]
        gsems = rest[_NBUF:2 * _NBUF]
        wsems = rest[2 * _NBUF:]
        wid = lax.axis_index("s") * nc + lax.axis_index("c")
        ubase = wid * per_w
        pltpu.sync_copy(idx_hbm.at[wid], idx_v)

        pb = _P.bit_length() - 1
        @pl.loop(0, per_w)
        def _(j):
            for k in range(_CHUNK // 16):
                v = idx_v[j, pl.ds(k * 16, 16)]
                g = jnp.right_shift(v, pb + 1)
                h = jnp.bitwise_and(jnp.right_shift(v, pb), 1)
                i2 = jnp.bitwise_and(v, _P - 1)
                idx_v[j, pl.ds(k * 16, 16)] = (
                    jnp.left_shift(g, pb + 1) + jnp.left_shift(i2, 1) + h)

        hg = [None] * _NBUF
        hw = [None] * _NBUF
        for j in range(per_w + _LAG):
            if j < per_w:
                b = j % _NBUF
                if j >= _NBUF:
                    hw[b].wait()
                hg[b] = pltpu.async_copy(
                    tab_hbm.at[idx_v.at[j]], bufs[b], gsems[b])
            i = j - _LAG
            if 0 <= i < per_w:
                bi = i % _NBUF
                hg[bi].wait()
                u = ubase + i
                li = jnp.right_shift(u, sh_l)
                b0 = jnp.left_shift(jnp.bitwise_and(u, bblks - 1), sh_b)
                hh = jnp.right_shift(b0, half.bit_length() - 1)
                row0 = pl.multiple_of(
                    li * half + jnp.bitwise_and(b0, half - 1), _CHUNK)
                col0 = pl.multiple_of(hh * DIM, DIM)
                hw[bi] = pltpu.async_copy(
                    bufs[bi],
                    out_hbm.at[pl.ds(row0, _CHUNK), pl.ds(col0, DIM)],
                    wsems[bi])
        for i in range(max(0, per_w - _NBUF), per_w):
            hw[i % _NBUF].wait()

    return gather_k


def _eye(n):
    return (jax.lax.broadcasted_iota(jnp.int32, (n, n), 0)
            == jax.lax.broadcasted_iota(jnp.int32, (n, n), 1)
            ).astype(jnp.float32)


_PL3 = 2


def _plane_body(f_ref, out_ref):
    x = f_ref[...]
    half = x.shape[0] // _PL3
    ident = _eye(DIM)
    dn = (((1,), (1,)), ((), ()))
    planes = []
    for p in range(_PL3):
        xp = x[p * half:(p + 1) * half]
        a = jax.lax.dot_general(ident, xp[:, :DIM], dn,
                                preferred_element_type=jnp.float32)
        b = jax.lax.dot_general(ident, xp[:, DIM:], dn,
                                preferred_element_type=jnp.float32)
        planes.append(jnp.concatenate([a, b], axis=1)[None])
    out_ref[...] = jnp.concatenate(planes, axis=0)


def _plane_transpose(f, seq, bsz):
    half = bsz // 2
    return pl.pallas_call(
        _plane_body,
        grid=(seq // _PL3,),
        in_specs=[pl.BlockSpec((_PL3 * half, 2 * DIM), lambda i: (i, 0))],
        out_specs=pl.BlockSpec((_PL3, DIM, bsz), lambda i: (i, 0, 0)),
        out_shape=jax.ShapeDtypeStruct((seq, DIM, bsz), jnp.float32),
    )(f)


def kernel(doc, table):
    b, l = doc.shape
    flat = _renorm_flat(table)
    nw = 32
    idx3d = doc.T.reshape(nw, b * l // (_CHUNK * nw), _CHUNK)
    f = _make_gather(b, l)(flat, idx3d)
    out = _plane_transpose(f, l, b)
    return jnp.transpose(out, (2, 0, 1))


# speedup vs baseline: 3.0273x; 1.1129x over previous
import functools

import jax
import jax.numpy as jnp
from jax import lax
from jax.experimental import pallas as pl
from jax.experimental.pallas import tpu as pltpu
from jax.experimental.pallas import tpu_sc as plsc

DIM = 64
MAX_NORM = 1.0


_P = 4096


def _renorm_body(tt_ref, out_ref):
    x = tt_ref[...]
    norm = jnp.sqrt(jnp.sum(x * x, axis=0, keepdims=True))
    scale = jnp.where(norm > MAX_NORM, MAX_NORM / (norm + 1e-7), 1.0)
    y = x * scale
    out_ref[...] = jnp.concatenate([y[:, :_P].T, y[:, _P:].T], axis=1)


def _renorm_flat(table):
    vocab = table.shape[0]
    g = (vocab + 2 * _P - 1) // (2 * _P)
    sf = pl.pallas_call(
        _renorm_body,
        grid=(g,),
        in_specs=[pl.BlockSpec((DIM, 2 * _P), lambda i: (0, i))],
        out_specs=pl.BlockSpec((_P, 2 * DIM), lambda i: (i, 0)),
        out_shape=jax.ShapeDtypeStruct((g * _P, 2 * DIM), jnp.float32),
    )(table.T)
    return sf.reshape(g * 2 * _P, DIM)


_CHUNK = 128
_NBUF = 6
_LAG = _NBUF // 2


@functools.cache
def _make_gather(bsz, seq):
    info = plsc.get_sparse_core_info()
    nc, ns = info.num_cores, info.num_subcores
    nw = nc * ns
    bblks = bsz // _CHUNK
    half = bsz // 2
    per_w = bblks * seq // nw
    assert per_w * nw == bblks * seq and bblks * _CHUNK == bsz
    assert bblks & (bblks - 1) == 0
    sh_l = bblks.bit_length() - 1
    sh_b = _CHUNK.bit_length() - 1
    mesh = plsc.VectorSubcoreMesh(core_axis_name="c", subcore_axis_name="s")

    @functools.partial(
        pl.kernel,
        mesh=mesh,
        compiler_params=pltpu.CompilerParams(
            use_tc_tiling_on_sc=False, needs_layout_passes=False),
        out_type=jax.ShapeDtypeStruct((seq * half, 2 * DIM), jnp.float32),
        scratch_types=(
            [pltpu.VMEM((per_w, _CHUNK), jnp.int32)]
            + [pltpu.VMEM((_CHUNK, DIM), jnp.float32) for _ in range(_NBUF)]
            + [pltpu.SemaphoreType.DMA for _ in range(2 * _NBUF)]
        ),
    )
    def gather_k(tab_hbm, idx_hbm, out_hbm, idx_v, *rest):
        bufs = rest[:_NBUF]
        gsems = rest[_NBUF:2 * _NBUF]
        wsems = rest[2 * _NBUF:]
        wid = lax.axis_index("s") * nc + lax.axis_index("c")
        ubase = wid * per_w
        pltpu.sync_copy(idx_hbm.at[wid], idx_v)

        pb = _P.bit_length() - 1
        @pl.loop(0, per_w)
        def _(j):
            for k in range(_CHUNK // 16):
                v = idx_v[j, pl.ds(k * 16, 16)]
                g = jnp.right_shift(v, pb + 1)
                h = jnp.bitwise_and(jnp.right_shift(v, pb), 1)
                i2 = jnp.bitwise_and(v, _P - 1)
                idx_v[j, pl.ds(k * 16, 16)] = (
                    jnp.left_shift(g, pb + 1) + jnp.left_shift(i2, 1) + h)

        hg = [None] * _NBUF
        hw = [None] * _NBUF
        for j in range(per_w + _LAG):
            if j < per_w:
                b = j % _NBUF
                if j >= _NBUF:
                    hw[b].wait()
                hg[b] = pltpu.async_copy(
                    tab_hbm.at[idx_v.at[j]], bufs[b], gsems[b])
            i = j - _LAG
            if 0 <= i < per_w:
                bi = i % _NBUF
                hg[bi].wait()
                u = ubase + i
                li = jnp.right_shift(u, sh_l)
                b0 = jnp.left_shift(jnp.bitwise_and(u, bblks - 1), sh_b)
                hh = jnp.right_shift(b0, half.bit_length() - 1)
                row0 = pl.multiple_of(
                    li * half + jnp.bitwise_and(b0, half - 1), _CHUNK)
                col0 = pl.multiple_of(hh * DIM, DIM)
                hw[bi] = pltpu.async_copy(
                    bufs[bi],
                    out_hbm.at[pl.ds(row0, _CHUNK), pl.ds(col0, DIM)],
                    wsems[bi])
        for i in range(max(0, per_w - _NBUF), per_w):
            hw[i % _NBUF].wait()

    return gather_k


def _eye(n):
    return (jax.lax.broadcasted_iota(jnp.int32, (n, n), 0)
            == jax.lax.broadcasted_iota(jnp.int32, (n, n), 1)
            ).astype(jnp.float32)


_PL3 = 5


def _plane_body(f_ref, out_ref):
    x = f_ref[...]
    half = x.shape[0] // _PL3
    ident = _eye(DIM)
    dn = (((1,), (1,)), ((), ()))
    planes = []
    for p in range(_PL3):
        xp = x[p * half:(p + 1) * half]
        a = jax.lax.dot_general(ident, xp[:, :DIM], dn,
                                preferred_element_type=jnp.float32)
        b = jax.lax.dot_general(ident, xp[:, DIM:], dn,
                                preferred_element_type=jnp.float32)
        planes.append(jnp.concatenate([a, b], axis=1)[None])
    out_ref[...] = jnp.concatenate(planes, axis=0)


def _plane_transpose(f, seq, bsz):
    half = bsz // 2
    return pl.pallas_call(
        _plane_body,
        grid=(seq // _PL3,),
        in_specs=[pl.BlockSpec((_PL3 * half, 2 * DIM), lambda i: (i, 0))],
        out_specs=pl.BlockSpec((_PL3, DIM, bsz), lambda i: (i, 0, 0)),
        out_shape=jax.ShapeDtypeStruct((seq, DIM, bsz), jnp.float32),
    )(f)


def kernel(doc, table):
    b, l = doc.shape
    flat = _renorm_flat(table)
    nw = 32
    idx3d = doc.T.reshape(nw, b * l // (_CHUNK * nw), _CHUNK)
    f = _make_gather(b, l)(flat, idx3d)
    out = _plane_transpose(f, l, b)
    return jnp.transpose(out, (2, 0, 1))
